# baseline (speedup 1.0000x reference)
import functools

import jax
import jax.numpy as jnp
from jax import lax
from jax.experimental import pallas as pl
from jax.experimental.pallas import tpu as pltpu
from jax.experimental.pallas import tpu_sc as plsc

N_NODES = 10000
N_EDGES = 160000
N_IDX = 2 * N_EDGES
NPAD = 10240
ROWS = NPAD // 128
RBLK = 16
NBLK = ROWS // RBLK
WIDE = RBLK * 128

_SC_MESH = plsc.VectorSubcoreMesh(core_axis_name="c", subcore_axis_name="s")
CBLK = 39 * 128
SLICE = NPAD // 16


@functools.partial(
    pl.kernel,
    mesh=_SC_MESH,
    out_type=jax.ShapeDtypeStruct((2 * NPAD,), jnp.float32),
    scratch_types=[
        pltpu.VMEM((2, CBLK), jnp.int32),
        pltpu.VMEM((2, 128), jnp.int32),
        pltpu.VMEM((NPAD,), jnp.float32),
        pltpu.VMEM((SLICE,), jnp.float32),
        pltpu.VMEM((SLICE,), jnp.float32),
        pltpu.VMEM_SHARED((16, NPAD), jnp.float32),
    ],
    compiler_params=pltpu.CompilerParams(needs_layout_passes=False),
)
def _sc_histogram(idx_hbm, t_hbm, idx_v, ext_v, hist_v, acc_v, buf_v, shared):
    c = lax.axis_index("c")
    s = lax.axis_index("s")
    w = c * 16 + s
    ones = jnp.ones((16,), jnp.float32)
    zeros16 = jnp.zeros((16,), jnp.float32)

    pltpu.sync_copy(idx_hbm.at[:, pl.ds(w * CBLK, CBLK)], idx_v)

    @pl.when(w < 2)
    def _():
        pltpu.sync_copy(idx_hbm.at[:, pl.ds((1248 + w) * 128, 128)], ext_v)

    def zero_body(i, _):
        hist_v[pl.ds(i * 16, 16)] = zeros16
        return 0
    lax.fori_loop(0, NPAD // 16, zero_body, 0, unroll=8)

    for r in range(2):
        def hist_body(i, _):
            vals = idx_v[r, pl.ds(i * 16, 16)]
            plsc.addupdate_scatter(hist_v, [vals], ones)
            return 0
        lax.fori_loop(0, CBLK // 16, hist_body, 0, unroll=4)

    @pl.when(w < 2)
    def _():
        for r in range(2):
            def ext_body(i, _):
                vals = ext_v[r, pl.ds(i * 16, 16)]
                plsc.addupdate_scatter(hist_v, [vals], ones)
                return 0
            lax.fori_loop(0, 8, ext_body, 0, unroll=4)

    pltpu.sync_copy(hist_v, shared.at[s])
    plsc.subcore_barrier()

    def acc_init(i, _):
        acc_v[pl.ds(i * 16, 16)] = zeros16
        return 0
    lax.fori_loop(0, SLICE // 16, acc_init, 0, unroll=8)

    def comb_body(t, _):
        pltpu.sync_copy(shared.at[t, pl.ds(s * SLICE, SLICE)], buf_v)

        def add_body(j, _):
            acc_v[pl.ds(j * 16, 16)] = acc_v[pl.ds(j * 16, 16)] + buf_v[pl.ds(j * 16, 16)]
            return 0
        lax.fori_loop(0, SLICE // 16, add_body, 0, unroll=8)
        return 0
    lax.fori_loop(0, 16, comb_body, 0)

    pltpu.sync_copy(acc_v, t_hbm.at[pl.ds(c * NPAD + s * SLICE, SLICE)])


def _dotT(a, b):
    return lax.dot_general(a, b, (((0,), (0,)), ((), ())),
                           precision=lax.Precision.HIGHEST,
                           preferred_element_type=jnp.float32)


def _tc_body(x_ref, w1r_ref, w1o_ref, b1_ref, w2r_ref, w2o_ref, b2_ref,
             wc_ref, bc_ref, ta_ref, tb_ref, out_ref, acc, tw):
    k = pl.program_id(0)
    relu = lambda a: jnp.maximum(a, 0.0)

    x = x_ref[...]
    w2o = w2o_ref[...]
    b1c = jnp.transpose(b1_ref[...])
    b2c = jnp.transpose(b2_ref[...])
    uT = _dotT(w1r_ref[...], jnp.transpose(x))
    vT = b1c + _dotT(w1o_ref[...], jnp.transpose(x))
    rvT = relu(vT)
    pT = _dotT(w2r_ref[...], rvT)

    t_sum = ta_ref[...] + tb_ref[...]
    for rr in range(RBLK):
        tw[0:1, rr * 128:(rr + 1) * 128] = t_sum[rr:rr + 1, :]
    t_wide = tw[...]

    node = k * WIDE + lax.broadcasted_iota(jnp.int32, (1, WIDE), 1)
    mask = (node < N_NODES).astype(jnp.float32)

    y2 = relu(uT[:, 0:1] * t_wide + vT[:, 2:3])
    y3 = relu(uT[:, 1:2] * t_wide + vT[:, 3:4])
    z2 = relu(pT[:, 0:1] * t_wide + b2c + _dotT(w2o, y2)) * mask
    z3 = relu(pT[:, 1:2] * t_wide + b2c + _dotT(w2o, y3)) * mask

    @pl.when(k == 0)
    def _():
        acc[...] = jnp.zeros_like(acc)

    acc[...] += jnp.concatenate(
        [jnp.sum(z2, axis=1, keepdims=True), jnp.sum(z3, axis=1, keepdims=True)], axis=1)

    @pl.when(k == NBLK - 1)
    def _():
        g01 = relu(b2c + _dotT(w2o, rvT[:, 0:2]))
        g = jnp.concatenate([g01, acc[...] * (1.0 / N_NODES)], axis=1)
        out_ref[...] = _dotT(g, wc_ref[...]) + bc_ref[...]


def _tc_head(x, W1_rel, W1_root, b1, W2_rel, W2_root, b2, Wc, bc, t2):
    full = lambda shape: pl.BlockSpec(shape, lambda k: (0, 0))
    return pl.pallas_call(
        _tc_body,
        grid=(NBLK,),
        in_specs=[
            full((4, 128)), full((128, 64)), full((128, 64)), full((1, 64)),
            full((64, 64)), full((64, 64)), full((1, 64)),
            full((64, 2)), full((1, 2)),
            pl.BlockSpec((RBLK, 128), lambda k: (k, 0)),
            pl.BlockSpec((RBLK, 128), lambda k: (NBLK + k, 0)),
        ],
        out_specs=full((4, 2)),
        out_shape=jax.ShapeDtypeStruct((4, 2), jnp.float32),
        scratch_shapes=[pltpu.VMEM((64, 2), jnp.float32),
                        pltpu.VMEM((1, WIDE), jnp.float32)],
    )(x, W1_rel, W1_root, b1, W2_rel, W2_root, b2, Wc, bc, t2, t2)


def kernel(x, edge_index, W1_rel, W1_root, b1, W2_rel, W2_root, b2, Wc, bc):
    t2 = _sc_histogram(edge_index).reshape(2 * ROWS, 128)
    return _tc_head(x, W1_rel, W1_root, b1.reshape(1, 64), W2_rel, W2_root,
                    b2.reshape(1, 64), Wc, bc.reshape(1, 2), t2)


# speedup vs baseline: 1.1103x; 1.1103x over previous
import functools

import jax
import jax.numpy as jnp
from jax import lax
from jax.experimental import pallas as pl
from jax.experimental.pallas import tpu as pltpu
from jax.experimental.pallas import tpu_sc as plsc

N_NODES = 10000
N_EDGES = 160000
N_IDX = 2 * N_EDGES
NPAD = 10240
ROWS = NPAD // 128

_SC_MESH = plsc.VectorSubcoreMesh(core_axis_name="c", subcore_axis_name="s")
CBLK = 39 * 128
SLICE = NPAD // 16


@functools.partial(
    pl.kernel,
    mesh=_SC_MESH,
    out_type=jax.ShapeDtypeStruct((2 * NPAD,), jnp.float32),
    scratch_types=[
        pltpu.VMEM((2, CBLK), jnp.int32),
        pltpu.VMEM((2, 128), jnp.int32),
        pltpu.VMEM((NPAD,), jnp.float32),
        pltpu.VMEM((SLICE,), jnp.float32),
        pltpu.VMEM((SLICE,), jnp.float32),
        pltpu.VMEM_SHARED((16, NPAD), jnp.float32),
        pltpu.SemaphoreType.DMA,
    ],
    compiler_params=pltpu.CompilerParams(needs_layout_passes=False),
)
def _sc_histogram(idx_hbm, t_hbm, idx_v, ext_v, hist_v, acc_v, buf_v, shared, sem):
    c = lax.axis_index("c")
    s = lax.axis_index("s")
    w = c * 16 + s
    ones = jnp.ones((16,), jnp.float32)
    zeros16 = jnp.zeros((16,), jnp.float32)

    cp = pltpu.async_copy(idx_hbm.at[:, pl.ds(w * CBLK, CBLK)], idx_v, sem)

    @pl.when(w < 2)
    def _():
        pltpu.sync_copy(idx_hbm.at[:, pl.ds((1248 + w) * 128, 128)], ext_v)

    def zero_body(i, _):
        hist_v[pl.ds(i * 16, 16)] = zeros16
        return 0
    lax.fori_loop(0, NPAD // 16, zero_body, 0, unroll=8)
    cp.wait()

    for r in range(2):
        def hist_body(i, _):
            vals = idx_v[r, pl.ds(i * 16, 16)]
            plsc.addupdate_scatter(hist_v, [vals], ones)
            return 0
        lax.fori_loop(0, CBLK // 16, hist_body, 0, unroll=4)

    @pl.when(w < 2)
    def _():
        for r in range(2):
            def ext_body(i, _):
                vals = ext_v[r, pl.ds(i * 16, 16)]
                plsc.addupdate_scatter(hist_v, [vals], ones)
                return 0
            lax.fori_loop(0, 8, ext_body, 0, unroll=4)

    pltpu.sync_copy(hist_v, shared.at[s])
    plsc.subcore_barrier()

    def acc_init(i, _):
        acc_v[pl.ds(i * 16, 16)] = zeros16
        return 0
    lax.fori_loop(0, SLICE // 16, acc_init, 0, unroll=8)

    def comb_body(t, _):
        pltpu.sync_copy(shared.at[t, pl.ds(s * SLICE, SLICE)], buf_v)

        def add_body(j, _):
            acc_v[pl.ds(j * 16, 16)] = acc_v[pl.ds(j * 16, 16)] + buf_v[pl.ds(j * 16, 16)]
            return 0
        lax.fori_loop(0, SLICE // 16, add_body, 0, unroll=8)
        return 0
    lax.fori_loop(0, 16, comb_body, 0)

    pltpu.sync_copy(acc_v, t_hbm.at[pl.ds(c * NPAD + s * SLICE, SLICE)])


def _dotT(a, b):
    return lax.dot_general(a, b, (((0,), (0,)), ((), ())),
                           precision=lax.Precision.HIGHEST,
                           preferred_element_type=jnp.float32)


def _tc_body(x_ref, w1r_ref, w1o_ref, b1_ref, w2r_ref, w2o_ref, b2_ref,
             wc_ref, bc_ref, ta_ref, tb_ref, out_ref, tw):
    relu = lambda a: jnp.maximum(a, 0.0)

    x = x_ref[...]
    w2o = w2o_ref[...]
    b1c = jnp.transpose(b1_ref[...])
    b2c = jnp.transpose(b2_ref[...])
    uT = _dotT(w1r_ref[...], jnp.transpose(x))
    vT = b1c + _dotT(w1o_ref[...], jnp.transpose(x))
    rvT = relu(vT)
    pT = _dotT(w2r_ref[...], rvT)

    t_sum = ta_ref[...] + tb_ref[...]
    for rr in range(ROWS):
        tw[0:1, rr * 128:(rr + 1) * 128] = t_sum[rr:rr + 1, :]
    t_wide = tw[...]

    node = lax.broadcasted_iota(jnp.int32, (1, NPAD), 1)
    mask = (node < N_NODES).astype(jnp.float32)

    y2 = relu(uT[:, 0:1] * t_wide + vT[:, 2:3])
    y3 = relu(uT[:, 1:2] * t_wide + vT[:, 3:4])
    z2 = relu(pT[:, 0:1] * t_wide + b2c + _dotT(w2o, y2)) * mask
    z3 = relu(pT[:, 1:2] * t_wide + b2c + _dotT(w2o, y3)) * mask

    s23 = jnp.concatenate(
        [jnp.sum(z2, axis=1, keepdims=True), jnp.sum(z3, axis=1, keepdims=True)],
        axis=1) * (1.0 / N_NODES)
    g01 = relu(b2c + _dotT(w2o, rvT[:, 0:2]))
    g = jnp.concatenate([g01, s23], axis=1)
    out_ref[...] = _dotT(g, wc_ref[...]) + bc_ref[...]


def _tc_head(x, W1_rel, W1_root, b1, W2_rel, W2_root, b2, Wc, bc, t2):
    full = lambda shape: pl.BlockSpec(shape, lambda k: (0, 0))
    return pl.pallas_call(
        _tc_body,
        grid=(1,),
        in_specs=[
            full((4, 128)), full((128, 64)), full((128, 64)), full((1, 64)),
            full((64, 64)), full((64, 64)), full((1, 64)),
            full((64, 2)), full((1, 2)),
            pl.BlockSpec((ROWS, 128), lambda k: (0, 0)),
            pl.BlockSpec((ROWS, 128), lambda k: (1, 0)),
        ],
        out_specs=full((4, 2)),
        out_shape=jax.ShapeDtypeStruct((4, 2), jnp.float32),
        scratch_shapes=[pltpu.VMEM((1, NPAD), jnp.float32)],
    )(x, W1_rel, W1_root, b1, W2_rel, W2_root, b2, Wc, bc, t2, t2)


def kernel(x, edge_index, W1_rel, W1_root, b1, W2_rel, W2_root, b2, Wc, bc):
    t2 = _sc_histogram(edge_index).reshape(2 * ROWS, 128)
    return _tc_head(x, W1_rel, W1_root, b1.reshape(1, 64), W2_rel, W2_root,
                    b2.reshape(1, 64), Wc, bc.reshape(1, 2), t2)
